# Initial kernel scaffold; baseline (speedup 1.0000x reference)
import functools

import jax
import jax.numpy as jnp
from jax import lax
from jax.experimental import pallas as pl
from jax.experimental.pallas import tpu as pltpu
from jax.experimental.pallas import tpu_sc as plsc

N = 10000
E = 320000
D_IN = 128
HID = 16
NUM_CLASSES = 64

NC = 2
NS = 16
NW = NC * NS
EPW = E // NW
CHUNK = 80
NCHUNK = EPW // CHUNK
ROWS_PER_TILE = N // NS

_mesh = plsc.VectorSubcoreMesh(core_axis_name="c", subcore_axis_name="s")


@functools.partial(
    pl.kernel,
    out_type=jax.ShapeDtypeStruct((NC, N), jnp.float32),
    mesh=_mesh,
    scratch_types=[
        pltpu.VMEM((NCHUNK, CHUNK), jnp.int32),
        pltpu.VMEM((CHUNK,), jnp.float32),
        pltpu.VMEM_SHARED((N,), jnp.float32),
    ],
)
def _deg_sc(dst_hbm, zeros1_hbm, out_hbm, dst_v, ones_v, deg_sh):
    c = lax.axis_index("c")
    s = lax.axis_index("s")
    for i in range(CHUNK // 16):
        ones_v[pl.ds(i * 16, 16)] = jnp.ones((16,), jnp.float32)

    @pl.when(s == 0)
    def _zero():
        pltpu.sync_copy(zeros1_hbm, deg_sh)

    plsc.subcore_barrier()
    base = (c * NS + s) * NCHUNK
    pltpu.sync_copy(dst_hbm.at[pl.ds(base, NCHUNK)], dst_v)

    def body(j, carry):
        pltpu.sync_copy(ones_v, deg_sh.at[dst_v.at[j]], add=True)
        return carry

    lax.fori_loop(0, NCHUNK, body, 0)
    plsc.subcore_barrier()

    @pl.when(s == 0)
    def _out():
        pltpu.sync_copy(deg_sh, out_hbm.at[c])


@functools.partial(
    pl.kernel,
    out_type=jax.ShapeDtypeStruct((NC, N, HID), jnp.float32),
    mesh=_mesh,
    scratch_types=[
        pltpu.VMEM((NCHUNK, CHUNK), jnp.int32),
        pltpu.VMEM((NCHUNK, CHUNK), jnp.int32),
        pltpu.VMEM((CHUNK, HID), jnp.float32),
        pltpu.VMEM_SHARED((N, HID), jnp.float32),
        pltpu.SemaphoreType.DMA,
    ],
)
def _agg_sc(hs_hbm, src_hbm, dst_hbm, zeros2_hbm, out_hbm,
            src_v, dst_v, rows_v, agg_sh, sem):
    c = lax.axis_index("c")
    s = lax.axis_index("s")
    rbase = s * ROWS_PER_TILE
    pltpu.sync_copy(zeros2_hbm.at[pl.ds(rbase, ROWS_PER_TILE)],
                    agg_sh.at[pl.ds(rbase, ROWS_PER_TILE)])
    plsc.subcore_barrier()
    base = (c * NS + s) * NCHUNK
    pltpu.sync_copy(src_hbm.at[pl.ds(base, NCHUNK)], src_v)
    pltpu.sync_copy(dst_hbm.at[pl.ds(base, NCHUNK)], dst_v)

    def body(j, carry):
        pltpu.async_copy(hs_hbm.at[src_v.at[j]], rows_v, sem).wait()
        pltpu.sync_copy(rows_v, agg_sh.at[dst_v.at[j]], add=True)
        return carry

    lax.fori_loop(0, NCHUNK, body, 0)
    plsc.subcore_barrier()
    pltpu.sync_copy(agg_sh.at[pl.ds(rbase, ROWS_PER_TILE)],
                    out_hbm.at[c, pl.ds(rbase, ROWS_PER_TILE)])


_RB = 1000
_GRID = N // _RB


def _s1_body(degp_ref, x_ref, w1_ref, hs_ref, dis_ref):
    deg = degp_ref[:, 0] + degp_ref[:, 1] + 1.0
    dis = lax.rsqrt(deg)
    h = jnp.dot(x_ref[...], w1_ref[...], preferred_element_type=jnp.float32)
    hs_ref[...] = h * dis[:, None]
    dis_ref[...] = dis[:, None]


def _tc_s1(degp_t, x, W1):
    return pl.pallas_call(
        _s1_body,
        grid=(_GRID,),
        in_specs=[
            pl.BlockSpec((_RB, NC), lambda i: (i, 0)),
            pl.BlockSpec((_RB, D_IN), lambda i: (i, 0)),
            pl.BlockSpec((D_IN, HID), lambda i: (0, 0)),
        ],
        out_specs=[
            pl.BlockSpec((_RB, HID), lambda i: (i, 0)),
            pl.BlockSpec((_RB, 1), lambda i: (i, 0)),
        ],
        out_shape=[
            jax.ShapeDtypeStruct((N, HID), jnp.float32),
            jax.ShapeDtypeStruct((N, 1), jnp.float32),
        ],
    )(degp_t, x, W1)


def _mid_body(a0_ref, a1_ref, hs1_ref, dis_ref, b1_ref, hs2_ref):
    dis = dis_ref[...]
    h1 = dis * (a0_ref[...] + a1_ref[...] + hs1_ref[...]) + b1_ref[...]
    hs2_ref[...] = dis * jnp.maximum(h1, 0.0)


def _tc_mid(a0, a1, hs1, dis, b1):
    return pl.pallas_call(
        _mid_body,
        grid=(_GRID,),
        in_specs=[
            pl.BlockSpec((_RB, HID), lambda i: (i, 0)),
            pl.BlockSpec((_RB, HID), lambda i: (i, 0)),
            pl.BlockSpec((_RB, HID), lambda i: (i, 0)),
            pl.BlockSpec((_RB, 1), lambda i: (i, 0)),
            pl.BlockSpec((1, HID), lambda i: (0, 0)),
        ],
        out_specs=pl.BlockSpec((_RB, HID), lambda i: (i, 0)),
        out_shape=jax.ShapeDtypeStruct((N, HID), jnp.float32),
    )(a0, a1, hs1, dis, b1)


def _out_body(a0_ref, a1_ref, hs2_ref, dis_ref, w2_ref, b2_ref, out_ref):
    a = dis_ref[...] * (a0_ref[...] + a1_ref[...] + hs2_ref[...])
    o = jnp.dot(a, w2_ref[...], preferred_element_type=jnp.float32)
    o = o + b2_ref[...]
    m = jnp.max(o, axis=1, keepdims=True)
    e = jnp.exp(o - m)
    lse = jnp.log(jnp.sum(e, axis=1, keepdims=True))
    out_ref[...] = (o - m) - lse


def _tc_out(a0, a1, hs2, dis, W2, b2):
    return pl.pallas_call(
        _out_body,
        grid=(_GRID,),
        in_specs=[
            pl.BlockSpec((_RB, HID), lambda i: (i, 0)),
            pl.BlockSpec((_RB, HID), lambda i: (i, 0)),
            pl.BlockSpec((_RB, HID), lambda i: (i, 0)),
            pl.BlockSpec((_RB, 1), lambda i: (i, 0)),
            pl.BlockSpec((HID, NUM_CLASSES), lambda i: (0, 0)),
            pl.BlockSpec((1, NUM_CLASSES), lambda i: (0, 0)),
        ],
        out_specs=pl.BlockSpec((_RB, NUM_CLASSES), lambda i: (i, 0)),
        out_shape=jax.ShapeDtypeStruct((N, NUM_CLASSES), jnp.float32),
    )(a0, a1, hs2, dis, W2, b2)


def kernel(x, edge_index, W1, b1, W2, b2):
    src2d = edge_index[0].reshape(E // CHUNK, CHUNK)
    dst2d = edge_index[1].reshape(E // CHUNK, CHUNK)
    zeros1 = jnp.zeros((N,), jnp.float32)
    zeros2 = jnp.zeros((N, HID), jnp.float32)

    degp = _deg_sc(dst2d, zeros1)
    hs1, dis = _tc_s1(degp.T, x, W1)
    agg1 = _agg_sc(hs1, src2d, dst2d, zeros2)
    hs2 = _tc_mid(agg1[0], agg1[1], hs1, dis, b1.reshape(1, HID))
    agg2 = _agg_sc(hs2, src2d, dst2d, zeros2)
    return _tc_out(agg2[0], agg2[1], hs2, dis, W2, b2.reshape(1, NUM_CLASSES))


# speedup vs baseline: 34.2178x; 34.2178x over previous
import functools

import jax
import jax.numpy as jnp
from jax import lax
from jax.experimental import pallas as pl
from jax.experimental.pallas import tpu as pltpu
from jax.experimental.pallas import tpu_sc as plsc

N = 10000
E = 320000
D_IN = 128
HID = 16
NUM_CLASSES = 64

NC = 2
NS = 16
NW = NC * NS
EPW = E // NW
CHUNK = 125
NCHUNK = EPW // CHUNK
OUT_TILES = 10
OUT_ROWS = N // OUT_TILES

_mesh = plsc.VectorSubcoreMesh(
    core_axis_name="c", subcore_axis_name="s", num_cores=NC, num_subcores=NS)


@functools.partial(
    pl.kernel,
    out_type=jax.ShapeDtypeStruct((NC, N), jnp.float32),
    mesh=_mesh,
    scratch_types=[
        pltpu.VMEM((NCHUNK, CHUNK), jnp.int32),
        pltpu.VMEM((128,), jnp.float32),
        pltpu.VMEM_SHARED((N,), jnp.float32),
    ],
)
def _deg_sc(dst_hbm, zeros1_hbm, out_hbm, dst_v, ones_v, deg_sh):
    c = lax.axis_index("c")
    s = lax.axis_index("s")
    for i in range(128 // 16):
        ones_v[pl.ds(i * 16, 16)] = jnp.ones((16,), jnp.float32)

    @pl.when(s == 0)
    def _zero():
        pltpu.sync_copy(zeros1_hbm, deg_sh)

    plsc.subcore_barrier()
    base = (c * NS + s) * NCHUNK
    pltpu.sync_copy(dst_hbm.at[pl.ds(base, NCHUNK)], dst_v)

    def body(j, carry):
        pltpu.sync_copy(ones_v.at[pl.ds(0, CHUNK)], deg_sh.at[dst_v.at[j]],
                        add=True)
        return carry

    lax.fori_loop(0, NCHUNK, body, 0)
    plsc.subcore_barrier()

    @pl.when(s == 0)
    def _out():
        pltpu.sync_copy(deg_sh, out_hbm.at[c])


@functools.partial(
    pl.kernel,
    out_type=jax.ShapeDtypeStruct((NC, N, HID), jnp.float32),
    mesh=_mesh,
    scratch_types=[
        pltpu.VMEM((NCHUNK, CHUNK), jnp.int32),
        pltpu.VMEM((NCHUNK, CHUNK), jnp.int32),
        pltpu.VMEM((CHUNK, HID), jnp.float32),
        pltpu.VMEM_SHARED((N, HID), jnp.float32),
        pltpu.SemaphoreType.DMA,
    ],
    compiler_params=pltpu.CompilerParams(use_tc_tiling_on_sc=False),
)
def _agg_sc(hs_hbm, src_hbm, dst_hbm, zeros2_hbm, out_hbm,
            src_v, dst_v, rows_v, agg_sh, sem):
    c = lax.axis_index("c")
    s = lax.axis_index("s")
    rbase = s * OUT_ROWS

    @pl.when(s < OUT_TILES)
    def _zero():
        pltpu.sync_copy(zeros2_hbm.at[pl.ds(rbase, OUT_ROWS)],
                        agg_sh.at[pl.ds(rbase, OUT_ROWS)])

    plsc.subcore_barrier()
    base = (c * NS + s) * NCHUNK
    pltpu.sync_copy(src_hbm.at[pl.ds(base, NCHUNK)], src_v)
    pltpu.sync_copy(dst_hbm.at[pl.ds(base, NCHUNK)], dst_v)

    def body(j, carry):
        pltpu.async_copy(hs_hbm.at[src_v.at[j]], rows_v, sem).wait()
        pltpu.sync_copy(rows_v, agg_sh.at[dst_v.at[j]], add=True)
        return carry

    lax.fori_loop(0, NCHUNK, body, 0)
    plsc.subcore_barrier()

    @pl.when(s < OUT_TILES)
    def _out():
        pltpu.sync_copy(agg_sh.at[pl.ds(rbase, OUT_ROWS)],
                        out_hbm.at[c, pl.ds(rbase, OUT_ROWS)])


_RB = 1000
_GRID = N // _RB


def _s1_body(degp_ref, x_ref, w1_ref, hs_ref, dis_ref):
    deg = degp_ref[:, 0] + degp_ref[:, 1] + 1.0
    dis = lax.rsqrt(deg)
    h = jnp.dot(x_ref[...], w1_ref[...], preferred_element_type=jnp.float32)
    hs_ref[...] = h * dis[:, None]
    dis_ref[...] = dis[:, None]


def _tc_s1(degp_t, x, W1):
    return pl.pallas_call(
        _s1_body,
        grid=(_GRID,),
        in_specs=[
            pl.BlockSpec((_RB, NC), lambda i: (i, 0)),
            pl.BlockSpec((_RB, D_IN), lambda i: (i, 0)),
            pl.BlockSpec((D_IN, HID), lambda i: (0, 0)),
        ],
        out_specs=[
            pl.BlockSpec((_RB, HID), lambda i: (i, 0)),
            pl.BlockSpec((_RB, 1), lambda i: (i, 0)),
        ],
        out_shape=[
            jax.ShapeDtypeStruct((N, HID), jnp.float32),
            jax.ShapeDtypeStruct((N, 1), jnp.float32),
        ],
    )(degp_t, x, W1)


def _mid_body(a0_ref, a1_ref, hs1_ref, dis_ref, b1_ref, hs2_ref):
    dis = dis_ref[...]
    h1 = dis * (a0_ref[...] + a1_ref[...] + hs1_ref[...]) + b1_ref[...]
    hs2_ref[...] = dis * jnp.maximum(h1, 0.0)


def _tc_mid(a0, a1, hs1, dis, b1):
    return pl.pallas_call(
        _mid_body,
        grid=(_GRID,),
        in_specs=[
            pl.BlockSpec((_RB, HID), lambda i: (i, 0)),
            pl.BlockSpec((_RB, HID), lambda i: (i, 0)),
            pl.BlockSpec((_RB, HID), lambda i: (i, 0)),
            pl.BlockSpec((_RB, 1), lambda i: (i, 0)),
            pl.BlockSpec((1, HID), lambda i: (0, 0)),
        ],
        out_specs=pl.BlockSpec((_RB, HID), lambda i: (i, 0)),
        out_shape=jax.ShapeDtypeStruct((N, HID), jnp.float32),
    )(a0, a1, hs1, dis, b1)


def _out_body(a0_ref, a1_ref, hs2_ref, dis_ref, w2_ref, b2_ref, out_ref):
    a = dis_ref[...] * (a0_ref[...] + a1_ref[...] + hs2_ref[...])
    o = jnp.dot(a, w2_ref[...], preferred_element_type=jnp.float32)
    o = o + b2_ref[...]
    m = jnp.max(o, axis=1, keepdims=True)
    e = jnp.exp(o - m)
    lse = jnp.log(jnp.sum(e, axis=1, keepdims=True))
    out_ref[...] = (o - m) - lse


def _tc_out(a0, a1, hs2, dis, W2, b2):
    return pl.pallas_call(
        _out_body,
        grid=(_GRID,),
        in_specs=[
            pl.BlockSpec((_RB, HID), lambda i: (i, 0)),
            pl.BlockSpec((_RB, HID), lambda i: (i, 0)),
            pl.BlockSpec((_RB, HID), lambda i: (i, 0)),
            pl.BlockSpec((_RB, 1), lambda i: (i, 0)),
            pl.BlockSpec((HID, NUM_CLASSES), lambda i: (0, 0)),
            pl.BlockSpec((1, NUM_CLASSES), lambda i: (0, 0)),
        ],
        out_specs=pl.BlockSpec((_RB, NUM_CLASSES), lambda i: (i, 0)),
        out_shape=jax.ShapeDtypeStruct((N, NUM_CLASSES), jnp.float32),
    )(a0, a1, hs2, dis, W2, b2)


def kernel(x, edge_index, W1, b1, W2, b2):
    src2d = edge_index[0].reshape(E // CHUNK, CHUNK)
    dst2d = edge_index[1].reshape(E // CHUNK, CHUNK)
    zeros1 = jnp.zeros((N,), jnp.float32)
    zeros2 = jnp.zeros((N, HID), jnp.float32)

    degp = _deg_sc(dst2d, zeros1)
    hs1, dis = _tc_s1(degp.T, x, W1)
    agg1 = _agg_sc(hs1, src2d, dst2d, zeros2)
    hs2 = _tc_mid(agg1[0], agg1[1], hs1, dis, b1.reshape(1, HID))
    agg2 = _agg_sc(hs2, src2d, dst2d, zeros2)
    return _tc_out(agg2[0], agg2[1], hs2, dis, W2, b2.reshape(1, NUM_CLASSES))


# speedup vs baseline: 52.5175x; 1.5348x over previous
import functools

import jax
import jax.numpy as jnp
from jax import lax
from jax.experimental import pallas as pl
from jax.experimental.pallas import tpu as pltpu
from jax.experimental.pallas import tpu_sc as plsc

N = 10000
E = 320000
D_IN = 128
HID = 16
NUM_CLASSES = 64

NC = 2
NS = 16
NW = NC * NS
EPW = E // NW
CHUNK = 125
NCHUNK = EPW // CHUNK
OUT_TILES = 10
OUT_ROWS = N // OUT_TILES
NBUF = 4

_mesh = plsc.VectorSubcoreMesh(
    core_axis_name="c", subcore_axis_name="s", num_cores=NC, num_subcores=NS)


@functools.partial(
    pl.kernel,
    out_type=jax.ShapeDtypeStruct((NC, N), jnp.float32),
    mesh=_mesh,
    scratch_types=[
        pltpu.VMEM((NCHUNK, CHUNK), jnp.int32),
        pltpu.VMEM((128,), jnp.float32),
        pltpu.VMEM_SHARED((N,), jnp.float32),
    ],
)
def _deg_sc(dst_hbm, zeros1_hbm, out_hbm, dst_v, ones_v, deg_sh):
    c = lax.axis_index("c")
    s = lax.axis_index("s")
    for i in range(128 // 16):
        ones_v[pl.ds(i * 16, 16)] = jnp.ones((16,), jnp.float32)

    @pl.when(s == 0)
    def _zero():
        pltpu.sync_copy(zeros1_hbm, deg_sh)

    plsc.subcore_barrier()
    base = (c * NS + s) * NCHUNK
    pltpu.sync_copy(dst_hbm.at[pl.ds(base, NCHUNK)], dst_v)

    def body(j, carry):
        pltpu.sync_copy(ones_v.at[pl.ds(0, CHUNK)], deg_sh.at[dst_v.at[j]],
                        add=True)
        return carry

    lax.fori_loop(0, NCHUNK, body, 0)
    plsc.subcore_barrier()

    @pl.when(s == 0)
    def _out():
        pltpu.sync_copy(deg_sh, out_hbm.at[c])


@functools.partial(
    pl.kernel,
    out_type=jax.ShapeDtypeStruct((NC, N, HID), jnp.float32),
    mesh=_mesh,
    scratch_types=[
        pltpu.VMEM((NCHUNK, CHUNK), jnp.int32),
        pltpu.VMEM((NCHUNK, CHUNK), jnp.int32),
        pltpu.VMEM((NBUF, CHUNK, HID), jnp.float32),
        pltpu.VMEM_SHARED((N, HID), jnp.float32),
        [pltpu.SemaphoreType.DMA] * NBUF,
        [pltpu.SemaphoreType.DMA] * NBUF,
    ],
    compiler_params=pltpu.CompilerParams(use_tc_tiling_on_sc=False),
)
def _agg_sc(hs_hbm, src_hbm, dst_hbm, zeros2_hbm, out_hbm,
            src_v, dst_v, rows_v, agg_sh, gsems, ssems):
    c = lax.axis_index("c")
    s = lax.axis_index("s")
    rbase = s * OUT_ROWS

    @pl.when(s < OUT_TILES)
    def _zero():
        pltpu.sync_copy(zeros2_hbm.at[pl.ds(rbase, OUT_ROWS)],
                        agg_sh.at[pl.ds(rbase, OUT_ROWS)])

    plsc.subcore_barrier()
    base = (c * NS + s) * NCHUNK
    pltpu.sync_copy(src_hbm.at[pl.ds(base, NCHUNK)], src_v)
    pltpu.sync_copy(dst_hbm.at[pl.ds(base, NCHUNK)], dst_v)

    for b in range(NBUF):
        pltpu.async_copy(hs_hbm.at[src_v.at[b]], rows_v.at[b], gsems[b])

    def body(o, carry):
        for b in range(NBUF):
            m = o * NBUF + b
            pltpu.make_async_copy(
                hs_hbm.at[src_v.at[m]], rows_v.at[b], gsems[b]).wait()
            pltpu.async_copy(
                rows_v.at[b], agg_sh.at[dst_v.at[m]], ssems[b], add=True)

            @pl.when(o < NCHUNK // NBUF - 1)
            def _next():
                pltpu.make_async_copy(
                    rows_v.at[b], agg_sh.at[dst_v.at[m]], ssems[b]).wait()
                pltpu.async_copy(
                    hs_hbm.at[src_v.at[m + NBUF]], rows_v.at[b], gsems[b])

        return carry

    lax.fori_loop(0, NCHUNK // NBUF, body, 0)
    for b in range(NBUF):
        m = NCHUNK - NBUF + b
        pltpu.make_async_copy(
            rows_v.at[b], agg_sh.at[dst_v.at[m]], ssems[b]).wait()
    plsc.subcore_barrier()

    @pl.when(s < OUT_TILES)
    def _out():
        pltpu.sync_copy(agg_sh.at[pl.ds(rbase, OUT_ROWS)],
                        out_hbm.at[c, pl.ds(rbase, OUT_ROWS)])


_RB = 1000
_GRID = N // _RB


def _s1_body(degp_ref, x_ref, w1_ref, hs_ref, dis_ref):
    deg = degp_ref[:, 0] + degp_ref[:, 1] + 1.0
    dis = lax.rsqrt(deg)
    h = jnp.dot(x_ref[...], w1_ref[...], preferred_element_type=jnp.float32)
    hs_ref[...] = h * dis[:, None]
    dis_ref[...] = dis[:, None]


def _tc_s1(degp_t, x, W1):
    return pl.pallas_call(
        _s1_body,
        grid=(_GRID,),
        in_specs=[
            pl.BlockSpec((_RB, NC), lambda i: (i, 0)),
            pl.BlockSpec((_RB, D_IN), lambda i: (i, 0)),
            pl.BlockSpec((D_IN, HID), lambda i: (0, 0)),
        ],
        out_specs=[
            pl.BlockSpec((_RB, HID), lambda i: (i, 0)),
            pl.BlockSpec((_RB, 1), lambda i: (i, 0)),
        ],
        out_shape=[
            jax.ShapeDtypeStruct((N, HID), jnp.float32),
            jax.ShapeDtypeStruct((N, 1), jnp.float32),
        ],
    )(degp_t, x, W1)


def _mid_body(a0_ref, a1_ref, hs1_ref, dis_ref, b1_ref, hs2_ref):
    dis = dis_ref[...]
    h1 = dis * (a0_ref[...] + a1_ref[...] + hs1_ref[...]) + b1_ref[...]
    hs2_ref[...] = dis * jnp.maximum(h1, 0.0)


def _tc_mid(a0, a1, hs1, dis, b1):
    return pl.pallas_call(
        _mid_body,
        grid=(_GRID,),
        in_specs=[
            pl.BlockSpec((_RB, HID), lambda i: (i, 0)),
            pl.BlockSpec((_RB, HID), lambda i: (i, 0)),
            pl.BlockSpec((_RB, HID), lambda i: (i, 0)),
            pl.BlockSpec((_RB, 1), lambda i: (i, 0)),
            pl.BlockSpec((1, HID), lambda i: (0, 0)),
        ],
        out_specs=pl.BlockSpec((_RB, HID), lambda i: (i, 0)),
        out_shape=jax.ShapeDtypeStruct((N, HID), jnp.float32),
    )(a0, a1, hs1, dis, b1)


def _out_body(a0_ref, a1_ref, hs2_ref, dis_ref, w2_ref, b2_ref, out_ref):
    a = dis_ref[...] * (a0_ref[...] + a1_ref[...] + hs2_ref[...])
    o = jnp.dot(a, w2_ref[...], preferred_element_type=jnp.float32)
    o = o + b2_ref[...]
    m = jnp.max(o, axis=1, keepdims=True)
    e = jnp.exp(o - m)
    lse = jnp.log(jnp.sum(e, axis=1, keepdims=True))
    out_ref[...] = (o - m) - lse


def _tc_out(a0, a1, hs2, dis, W2, b2):
    return pl.pallas_call(
        _out_body,
        grid=(_GRID,),
        in_specs=[
            pl.BlockSpec((_RB, HID), lambda i: (i, 0)),
            pl.BlockSpec((_RB, HID), lambda i: (i, 0)),
            pl.BlockSpec((_RB, HID), lambda i: (i, 0)),
            pl.BlockSpec((_RB, 1), lambda i: (i, 0)),
            pl.BlockSpec((HID, NUM_CLASSES), lambda i: (0, 0)),
            pl.BlockSpec((1, NUM_CLASSES), lambda i: (0, 0)),
        ],
        out_specs=pl.BlockSpec((_RB, NUM_CLASSES), lambda i: (i, 0)),
        out_shape=jax.ShapeDtypeStruct((N, NUM_CLASSES), jnp.float32),
    )(a0, a1, hs2, dis, W2, b2)


def kernel(x, edge_index, W1, b1, W2, b2):
    src2d = edge_index[0].reshape(E // CHUNK, CHUNK)
    dst2d = edge_index[1].reshape(E // CHUNK, CHUNK)
    zeros1 = jnp.zeros((N,), jnp.float32)
    zeros2 = jnp.zeros((N, HID), jnp.float32)

    degp = _deg_sc(dst2d, zeros1)
    hs1, dis = _tc_s1(degp.T, x, W1)
    agg1 = _agg_sc(hs1, src2d, dst2d, zeros2)
    hs2 = _tc_mid(agg1[0], agg1[1], hs1, dis, b1.reshape(1, HID))
    agg2 = _agg_sc(hs2, src2d, dst2d, zeros2)
    return _tc_out(agg2[0], agg2[1], hs2, dis, W2, b2.reshape(1, NUM_CLASSES))


# speedup vs baseline: 59.2560x; 1.1283x over previous
import functools

import jax
import jax.numpy as jnp
from jax import lax
from jax.experimental import pallas as pl
from jax.experimental.pallas import tpu as pltpu
from jax.experimental.pallas import tpu_sc as plsc

N = 10000
E = 320000
D_IN = 128
HID = 16
NUM_CLASSES = 64

NC = 2
NS = 16
NW = NC * NS
EPW = E // NW
CHUNK = 125
NCHUNK = EPW // CHUNK
OUT_TILES = 10
OUT_ROWS = N // OUT_TILES
NBUF = 8
DEG_Q = 8

_mesh = plsc.VectorSubcoreMesh(
    core_axis_name="c", subcore_axis_name="s", num_cores=NC, num_subcores=NS)


@functools.partial(
    pl.kernel,
    out_type=jax.ShapeDtypeStruct((NC, N), jnp.float32),
    mesh=_mesh,
    scratch_types=[
        pltpu.VMEM((NCHUNK, CHUNK), jnp.int32),
        pltpu.VMEM((128,), jnp.float32),
        pltpu.VMEM_SHARED((N,), jnp.float32),
        pltpu.SemaphoreType.DMA,
    ],
)
def _deg_sc(dst_hbm, zeros1_hbm, out_hbm, dst_v, ones_v, deg_sh, sem):
    c = lax.axis_index("c")
    s = lax.axis_index("s")
    for i in range(128 // 16):
        ones_v[pl.ds(i * 16, 16)] = jnp.ones((16,), jnp.float32)

    @pl.when(s == 0)
    def _zero():
        pltpu.sync_copy(zeros1_hbm, deg_sh)

    plsc.subcore_barrier()
    base = (c * NS + s) * NCHUNK
    pltpu.sync_copy(dst_hbm.at[pl.ds(base, NCHUNK)], dst_v)

    def body(j, carry):
        pltpu.async_copy(ones_v.at[pl.ds(0, CHUNK)], deg_sh.at[dst_v.at[j]],
                         sem, add=True)

        @pl.when(j >= DEG_Q)
        def _pace():
            pltpu.make_async_copy(
                ones_v.at[pl.ds(0, CHUNK)], deg_sh.at[dst_v.at[j]], sem).wait()

        return carry

    lax.fori_loop(0, NCHUNK, body, 0)
    for _ in range(DEG_Q):
        pltpu.make_async_copy(
            ones_v.at[pl.ds(0, CHUNK)], deg_sh.at[dst_v.at[0]], sem).wait()
    plsc.subcore_barrier()

    @pl.when(s == 0)
    def _out():
        pltpu.sync_copy(deg_sh, out_hbm.at[c])


@functools.partial(
    pl.kernel,
    out_type=jax.ShapeDtypeStruct((NC, N, HID), jnp.float32),
    mesh=_mesh,
    scratch_types=[
        pltpu.VMEM((NCHUNK, CHUNK), jnp.int32),
        pltpu.VMEM((NCHUNK, CHUNK), jnp.int32),
        pltpu.VMEM((NBUF, CHUNK, HID), jnp.float32),
        pltpu.VMEM_SHARED((N, HID), jnp.float32),
        [pltpu.SemaphoreType.DMA] * NBUF,
        [pltpu.SemaphoreType.DMA] * NBUF,
    ],
    compiler_params=pltpu.CompilerParams(use_tc_tiling_on_sc=False),
)
def _agg_sc(hs_hbm, src_hbm, dst_hbm, zeros2_hbm, out_hbm,
            src_v, dst_v, rows_v, agg_sh, gsems, ssems):
    c = lax.axis_index("c")
    s = lax.axis_index("s")
    rbase = s * OUT_ROWS

    @pl.when(s < OUT_TILES)
    def _zero():
        pltpu.sync_copy(zeros2_hbm.at[pl.ds(rbase, OUT_ROWS)],
                        agg_sh.at[pl.ds(rbase, OUT_ROWS)])

    plsc.subcore_barrier()
    base = (c * NS + s) * NCHUNK
    pltpu.sync_copy(src_hbm.at[pl.ds(base, NCHUNK)], src_v)
    pltpu.sync_copy(dst_hbm.at[pl.ds(base, NCHUNK)], dst_v)

    for b in range(NBUF):
        pltpu.async_copy(hs_hbm.at[src_v.at[b]], rows_v.at[b], gsems[b])

    def body(o, carry):
        for b in range(NBUF):
            m = o * NBUF + b
            pltpu.make_async_copy(
                hs_hbm.at[src_v.at[m]], rows_v.at[b], gsems[b]).wait()
            pltpu.async_copy(
                rows_v.at[b], agg_sh.at[dst_v.at[m]], ssems[b], add=True)

            @pl.when(o < NCHUNK // NBUF - 1)
            def _next():
                pltpu.make_async_copy(
                    rows_v.at[b], agg_sh.at[dst_v.at[m]], ssems[b]).wait()
                pltpu.async_copy(
                    hs_hbm.at[src_v.at[m + NBUF]], rows_v.at[b], gsems[b])

        return carry

    lax.fori_loop(0, NCHUNK // NBUF, body, 0)
    for b in range(NBUF):
        m = NCHUNK - NBUF + b
        pltpu.make_async_copy(
            rows_v.at[b], agg_sh.at[dst_v.at[m]], ssems[b]).wait()
    plsc.subcore_barrier()

    @pl.when(s < OUT_TILES)
    def _out():
        pltpu.sync_copy(agg_sh.at[pl.ds(rbase, OUT_ROWS)],
                        out_hbm.at[c, pl.ds(rbase, OUT_ROWS)])


_RB = 1000
_GRID = N // _RB


def _s1_body(degp_ref, x_ref, w1_ref, hs_ref, dis_ref):
    deg = degp_ref[:, 0] + degp_ref[:, 1] + 1.0
    dis = lax.rsqrt(deg)
    h = jnp.dot(x_ref[...], w1_ref[...], preferred_element_type=jnp.float32)
    hs_ref[...] = h * dis[:, None]
    dis_ref[...] = dis[:, None]


def _tc_s1(degp_t, x, W1):
    return pl.pallas_call(
        _s1_body,
        grid=(_GRID,),
        in_specs=[
            pl.BlockSpec((_RB, NC), lambda i: (i, 0)),
            pl.BlockSpec((_RB, D_IN), lambda i: (i, 0)),
            pl.BlockSpec((D_IN, HID), lambda i: (0, 0)),
        ],
        out_specs=[
            pl.BlockSpec((_RB, HID), lambda i: (i, 0)),
            pl.BlockSpec((_RB, 1), lambda i: (i, 0)),
        ],
        out_shape=[
            jax.ShapeDtypeStruct((N, HID), jnp.float32),
            jax.ShapeDtypeStruct((N, 1), jnp.float32),
        ],
    )(degp_t, x, W1)


def _mid_body(a0_ref, a1_ref, hs1_ref, dis_ref, b1_ref, hs2_ref):
    dis = dis_ref[...]
    h1 = dis * (a0_ref[...] + a1_ref[...] + hs1_ref[...]) + b1_ref[...]
    hs2_ref[...] = dis * jnp.maximum(h1, 0.0)


def _tc_mid(a0, a1, hs1, dis, b1):
    return pl.pallas_call(
        _mid_body,
        grid=(_GRID,),
        in_specs=[
            pl.BlockSpec((_RB, HID), lambda i: (i, 0)),
            pl.BlockSpec((_RB, HID), lambda i: (i, 0)),
            pl.BlockSpec((_RB, HID), lambda i: (i, 0)),
            pl.BlockSpec((_RB, 1), lambda i: (i, 0)),
            pl.BlockSpec((1, HID), lambda i: (0, 0)),
        ],
        out_specs=pl.BlockSpec((_RB, HID), lambda i: (i, 0)),
        out_shape=jax.ShapeDtypeStruct((N, HID), jnp.float32),
    )(a0, a1, hs1, dis, b1)


def _out_body(a0_ref, a1_ref, hs2_ref, dis_ref, w2_ref, b2_ref, out_ref):
    a = dis_ref[...] * (a0_ref[...] + a1_ref[...] + hs2_ref[...])
    o = jnp.dot(a, w2_ref[...], preferred_element_type=jnp.float32)
    o = o + b2_ref[...]
    m = jnp.max(o, axis=1, keepdims=True)
    e = jnp.exp(o - m)
    lse = jnp.log(jnp.sum(e, axis=1, keepdims=True))
    out_ref[...] = (o - m) - lse


def _tc_out(a0, a1, hs2, dis, W2, b2):
    return pl.pallas_call(
        _out_body,
        grid=(_GRID,),
        in_specs=[
            pl.BlockSpec((_RB, HID), lambda i: (i, 0)),
            pl.BlockSpec((_RB, HID), lambda i: (i, 0)),
            pl.BlockSpec((_RB, HID), lambda i: (i, 0)),
            pl.BlockSpec((_RB, 1), lambda i: (i, 0)),
            pl.BlockSpec((HID, NUM_CLASSES), lambda i: (0, 0)),
            pl.BlockSpec((1, NUM_CLASSES), lambda i: (0, 0)),
        ],
        out_specs=pl.BlockSpec((_RB, NUM_CLASSES), lambda i: (i, 0)),
        out_shape=jax.ShapeDtypeStruct((N, NUM_CLASSES), jnp.float32),
    )(a0, a1, hs2, dis, W2, b2)


def kernel(x, edge_index, W1, b1, W2, b2):
    src2d = edge_index[0].reshape(E // CHUNK, CHUNK)
    dst2d = edge_index[1].reshape(E // CHUNK, CHUNK)
    zeros1 = jnp.zeros((N,), jnp.float32)
    zeros2 = jnp.zeros((N, HID), jnp.float32)

    degp = _deg_sc(dst2d, zeros1)
    hs1, dis = _tc_s1(degp.T, x, W1)
    agg1 = _agg_sc(hs1, src2d, dst2d, zeros2)
    hs2 = _tc_mid(agg1[0], agg1[1], hs1, dis, b1.reshape(1, HID))
    agg2 = _agg_sc(hs2, src2d, dst2d, zeros2)
    return _tc_out(agg2[0], agg2[1], hs2, dis, W2, b2.reshape(1, NUM_CLASSES))


# speedup vs baseline: 61.3923x; 1.0361x over previous
import functools

import jax
import jax.numpy as jnp
from jax import lax
from jax.experimental import pallas as pl
from jax.experimental.pallas import tpu as pltpu
from jax.experimental.pallas import tpu_sc as plsc

N = 10000
E = 320000
D_IN = 128
HID = 16
NUM_CLASSES = 64

NC = 2
NS = 16
NW = NC * NS
EPW = E // NW
CHUNK = 125
NCHUNK = EPW // CHUNK
OUT_TILES = 10
OUT_ROWS = N // OUT_TILES
NBUF = 8
DEG_Q = 8

_mesh = plsc.VectorSubcoreMesh(
    core_axis_name="c", subcore_axis_name="s", num_cores=NC, num_subcores=NS)


@functools.partial(
    pl.kernel,
    out_type=jax.ShapeDtypeStruct((NC, N), jnp.float32),
    mesh=_mesh,
    scratch_types=[
        pltpu.VMEM((NCHUNK, CHUNK), jnp.int32),
        pltpu.VMEM((128,), jnp.float32),
        pltpu.VMEM_SHARED((N,), jnp.float32),
        pltpu.SemaphoreType.DMA,
    ],
)
def _deg_sc(dst_hbm, zeros1_hbm, out_hbm, dst_v, ones_v, deg_sh, sem):
    c = lax.axis_index("c")
    s = lax.axis_index("s")
    for i in range(128 // 16):
        ones_v[pl.ds(i * 16, 16)] = jnp.ones((16,), jnp.float32)

    @pl.when(s == 0)
    def _zero():
        pltpu.sync_copy(zeros1_hbm, deg_sh)

    plsc.subcore_barrier()
    base = (c * NS + s) * NCHUNK
    pltpu.sync_copy(dst_hbm.at[pl.ds(base, NCHUNK)], dst_v)

    def body(j, carry):
        pltpu.async_copy(ones_v.at[pl.ds(0, CHUNK)], deg_sh.at[dst_v.at[j]],
                         sem, add=True)

        @pl.when(j >= DEG_Q)
        def _pace():
            pltpu.make_async_copy(
                ones_v.at[pl.ds(0, CHUNK)], deg_sh.at[dst_v.at[j]], sem).wait()

        return carry

    lax.fori_loop(0, NCHUNK, body, 0)
    for _ in range(DEG_Q):
        pltpu.make_async_copy(
            ones_v.at[pl.ds(0, CHUNK)], deg_sh.at[dst_v.at[0]], sem).wait()
    plsc.subcore_barrier()

    @pl.when(s == 0)
    def _out():
        pltpu.sync_copy(deg_sh, out_hbm.at[c])


@functools.partial(
    pl.kernel,
    out_type=jax.ShapeDtypeStruct((NC, N, HID), jnp.float32),
    mesh=_mesh,
    scratch_types=[
        pltpu.VMEM((NCHUNK, CHUNK), jnp.int32),
        pltpu.VMEM((NCHUNK, CHUNK), jnp.int32),
        pltpu.VMEM((NBUF, CHUNK, HID), jnp.float32),
        pltpu.VMEM_SHARED((N, HID), jnp.float32),
        [pltpu.SemaphoreType.DMA] * NBUF,
        [pltpu.SemaphoreType.DMA] * NBUF,
    ],
    compiler_params=pltpu.CompilerParams(use_tc_tiling_on_sc=False),
)
def _agg_sc(hs_hbm, src_hbm, dst_hbm, zeros2_hbm, out_hbm,
            src_v, dst_v, rows_v, agg_sh, gsems, ssems):
    c = lax.axis_index("c")
    s = lax.axis_index("s")
    rbase = s * OUT_ROWS

    @pl.when(s < OUT_TILES)
    def _zero():
        pltpu.sync_copy(zeros2_hbm.at[pl.ds(rbase, OUT_ROWS)],
                        agg_sh.at[pl.ds(rbase, OUT_ROWS)])

    plsc.subcore_barrier()
    base = (c * NS + s) * NCHUNK
    pltpu.sync_copy(src_hbm.at[pl.ds(base, NCHUNK)], src_v)
    pltpu.sync_copy(dst_hbm.at[pl.ds(base, NCHUNK)], dst_v)

    for b in range(NBUF):
        pltpu.async_copy(hs_hbm.at[src_v.at[b]], rows_v.at[b], gsems[b])

    def body(o, carry):
        for b in range(NBUF):
            m = o * NBUF + b
            pltpu.make_async_copy(
                hs_hbm.at[src_v.at[m]], rows_v.at[b], gsems[b]).wait()
            pltpu.async_copy(
                rows_v.at[b], agg_sh.at[dst_v.at[m]], ssems[b], add=True)

            @pl.when(o < NCHUNK // NBUF - 1)
            def _next():
                pltpu.make_async_copy(
                    rows_v.at[b], agg_sh.at[dst_v.at[m]], ssems[b]).wait()
                pltpu.async_copy(
                    hs_hbm.at[src_v.at[m + NBUF]], rows_v.at[b], gsems[b])

        return carry

    lax.fori_loop(0, NCHUNK // NBUF, body, 0)
    for b in range(NBUF):
        m = NCHUNK - NBUF + b
        pltpu.make_async_copy(
            rows_v.at[b], agg_sh.at[dst_v.at[m]], ssems[b]).wait()
    plsc.subcore_barrier()

    @pl.when(s < OUT_TILES)
    def _out():
        pltpu.sync_copy(agg_sh.at[pl.ds(rbase, OUT_ROWS)],
                        out_hbm.at[c, pl.ds(rbase, OUT_ROWS)])


_RB = N
_GRID = N // _RB


def _s1_body(degp_ref, x_ref, w1_ref, hs_ref, dis_ref):
    deg = degp_ref[:, 0] + degp_ref[:, 1] + 1.0
    dis = lax.rsqrt(deg)
    h = jnp.dot(x_ref[...], w1_ref[...], preferred_element_type=jnp.float32)
    hs_ref[...] = h * dis[:, None]
    dis_ref[...] = dis[:, None]


def _tc_s1(degp_t, x, W1):
    return pl.pallas_call(
        _s1_body,
        grid=(_GRID,),
        in_specs=[
            pl.BlockSpec((_RB, NC), lambda i: (i, 0)),
            pl.BlockSpec((_RB, D_IN), lambda i: (i, 0)),
            pl.BlockSpec((D_IN, HID), lambda i: (0, 0)),
        ],
        out_specs=[
            pl.BlockSpec((_RB, HID), lambda i: (i, 0)),
            pl.BlockSpec((_RB, 1), lambda i: (i, 0)),
        ],
        out_shape=[
            jax.ShapeDtypeStruct((N, HID), jnp.float32),
            jax.ShapeDtypeStruct((N, 1), jnp.float32),
        ],
    )(degp_t, x, W1)


def _mid_body(a0_ref, a1_ref, hs1_ref, dis_ref, b1_ref, hs2_ref):
    dis = dis_ref[...]
    h1 = dis * (a0_ref[...] + a1_ref[...] + hs1_ref[...]) + b1_ref[...]
    hs2_ref[...] = dis * jnp.maximum(h1, 0.0)


def _tc_mid(a0, a1, hs1, dis, b1):
    return pl.pallas_call(
        _mid_body,
        grid=(_GRID,),
        in_specs=[
            pl.BlockSpec((_RB, HID), lambda i: (i, 0)),
            pl.BlockSpec((_RB, HID), lambda i: (i, 0)),
            pl.BlockSpec((_RB, HID), lambda i: (i, 0)),
            pl.BlockSpec((_RB, 1), lambda i: (i, 0)),
            pl.BlockSpec((1, HID), lambda i: (0, 0)),
        ],
        out_specs=pl.BlockSpec((_RB, HID), lambda i: (i, 0)),
        out_shape=jax.ShapeDtypeStruct((N, HID), jnp.float32),
    )(a0, a1, hs1, dis, b1)


def _out_body(a0_ref, a1_ref, hs2_ref, dis_ref, w2_ref, b2_ref, out_ref):
    a = dis_ref[...] * (a0_ref[...] + a1_ref[...] + hs2_ref[...])
    o = jnp.dot(a, w2_ref[...], preferred_element_type=jnp.float32)
    o = o + b2_ref[...]
    m = jnp.max(o, axis=1, keepdims=True)
    e = jnp.exp(o - m)
    lse = jnp.log(jnp.sum(e, axis=1, keepdims=True))
    out_ref[...] = (o - m) - lse


def _tc_out(a0, a1, hs2, dis, W2, b2):
    return pl.pallas_call(
        _out_body,
        grid=(_GRID,),
        in_specs=[
            pl.BlockSpec((_RB, HID), lambda i: (i, 0)),
            pl.BlockSpec((_RB, HID), lambda i: (i, 0)),
            pl.BlockSpec((_RB, HID), lambda i: (i, 0)),
            pl.BlockSpec((_RB, 1), lambda i: (i, 0)),
            pl.BlockSpec((HID, NUM_CLASSES), lambda i: (0, 0)),
            pl.BlockSpec((1, NUM_CLASSES), lambda i: (0, 0)),
        ],
        out_specs=pl.BlockSpec((_RB, NUM_CLASSES), lambda i: (i, 0)),
        out_shape=jax.ShapeDtypeStruct((N, NUM_CLASSES), jnp.float32),
    )(a0, a1, hs2, dis, W2, b2)


def kernel(x, edge_index, W1, b1, W2, b2):
    src2d = edge_index[0].reshape(E // CHUNK, CHUNK)
    dst2d = edge_index[1].reshape(E // CHUNK, CHUNK)
    zeros1 = jnp.zeros((N,), jnp.float32)
    zeros2 = jnp.zeros((N, HID), jnp.float32)

    degp = _deg_sc(dst2d, zeros1)
    hs1, dis = _tc_s1(degp.T, x, W1)
    agg1 = _agg_sc(hs1, src2d, dst2d, zeros2)
    hs2 = _tc_mid(agg1[0], agg1[1], hs1, dis, b1.reshape(1, HID))
    agg2 = _agg_sc(hs2, src2d, dst2d, zeros2)
    return _tc_out(agg2[0], agg2[1], hs2, dis, W2, b2.reshape(1, NUM_CLASSES))


# speedup vs baseline: 64.4433x; 1.0497x over previous
import functools

import jax
import jax.numpy as jnp
from jax import lax
from jax.experimental import pallas as pl
from jax.experimental.pallas import tpu as pltpu
from jax.experimental.pallas import tpu_sc as plsc

N = 10000
E = 320000
D_IN = 128
HID = 16
NUM_CLASSES = 64

NC = 2
NS = 16
NW = NC * NS
EPW = E // NW
CHUNK = 125
NCHUNK = EPW // CHUNK
OUT_TILES = 10
OUT_ROWS = N // OUT_TILES
NBUF = 8
DEG_Q = 8

_mesh = plsc.VectorSubcoreMesh(
    core_axis_name="c", subcore_axis_name="s", num_cores=NC, num_subcores=NS)


@functools.partial(
    pl.kernel,
    out_type=jax.ShapeDtypeStruct((NC, N), jnp.float32),
    mesh=_mesh,
    scratch_types=[
        pltpu.VMEM((NCHUNK, CHUNK), jnp.int32),
        pltpu.VMEM((128,), jnp.float32),
        pltpu.VMEM_SHARED((N,), jnp.float32),
        pltpu.SemaphoreType.DMA,
    ],
)
def _deg_sc(dst_hbm, zeros1_hbm, out_hbm, dst_v, ones_v, deg_sh, sem):
    c = lax.axis_index("c")
    s = lax.axis_index("s")
    for i in range(128 // 16):
        ones_v[pl.ds(i * 16, 16)] = jnp.ones((16,), jnp.float32)

    @pl.when(s == 0)
    def _zero():
        pltpu.sync_copy(zeros1_hbm, deg_sh)

    plsc.subcore_barrier()
    base = (c * NS + s) * NCHUNK
    pltpu.sync_copy(dst_hbm.at[pl.ds(base, NCHUNK)], dst_v)

    def body(j, carry):
        pltpu.async_copy(ones_v.at[pl.ds(0, CHUNK)], deg_sh.at[dst_v.at[j]],
                         sem, add=True)

        @pl.when(j >= DEG_Q)
        def _pace():
            pltpu.make_async_copy(
                ones_v.at[pl.ds(0, CHUNK)], deg_sh.at[dst_v.at[j]], sem).wait()

        return carry

    lax.fori_loop(0, NCHUNK, body, 0)
    for _ in range(DEG_Q):
        pltpu.make_async_copy(
            ones_v.at[pl.ds(0, CHUNK)], deg_sh.at[dst_v.at[0]], sem).wait()
    plsc.subcore_barrier()

    @pl.when(s == 0)
    def _out():
        pltpu.sync_copy(deg_sh, out_hbm.at[c])


@functools.partial(
    pl.kernel,
    out_type=jax.ShapeDtypeStruct((NC, N, HID), jnp.float32),
    mesh=_mesh,
    scratch_types=[
        pltpu.VMEM((NCHUNK, CHUNK), jnp.int32),
        pltpu.VMEM((NCHUNK, CHUNK), jnp.int32),
        pltpu.VMEM((NBUF, CHUNK, HID), jnp.float32),
        pltpu.VMEM_SHARED((N, HID), jnp.float32),
        [pltpu.SemaphoreType.DMA] * NBUF,
        [pltpu.SemaphoreType.DMA] * NBUF,
    ],
    compiler_params=pltpu.CompilerParams(use_tc_tiling_on_sc=False),
)
def _agg_sc(hs_hbm, src_hbm, dst_hbm, zeros2_hbm, out_hbm,
            src_v, dst_v, rows_v, agg_sh, gsems, ssems):
    c = lax.axis_index("c")
    s = lax.axis_index("s")
    rbase = s * OUT_ROWS

    @pl.when(s < OUT_TILES)
    def _zero():
        pltpu.sync_copy(zeros2_hbm.at[pl.ds(rbase, OUT_ROWS)],
                        agg_sh.at[pl.ds(rbase, OUT_ROWS)])

    plsc.subcore_barrier()
    base = (c * NS + s) * NCHUNK
    pltpu.sync_copy(src_hbm.at[pl.ds(base, NCHUNK)], src_v)
    pltpu.sync_copy(dst_hbm.at[pl.ds(base, NCHUNK)], dst_v)

    for b in range(NBUF):
        pltpu.async_copy(hs_hbm.at[src_v.at[b]], rows_v.at[b], gsems[b])

    def body(o, carry):
        for b in range(NBUF):
            m = o * NBUF + b
            pltpu.make_async_copy(
                hs_hbm.at[src_v.at[m]], rows_v.at[b], gsems[b]).wait()
            pltpu.async_copy(
                rows_v.at[b], agg_sh.at[dst_v.at[m]], ssems[b], add=True)

            @pl.when(o < NCHUNK // NBUF - 1)
            def _next():
                pltpu.make_async_copy(
                    rows_v.at[b], agg_sh.at[dst_v.at[m]], ssems[b]).wait()
                pltpu.async_copy(
                    hs_hbm.at[src_v.at[m + NBUF]], rows_v.at[b], gsems[b])

        return carry

    lax.fori_loop(0, NCHUNK // NBUF, body, 0)
    for b in range(NBUF):
        m = NCHUNK - NBUF + b
        pltpu.make_async_copy(
            rows_v.at[b], agg_sh.at[dst_v.at[m]], ssems[b]).wait()
    plsc.subcore_barrier()

    @pl.when(s < OUT_TILES)
    def _out():
        pltpu.sync_copy(agg_sh.at[pl.ds(rbase, OUT_ROWS)],
                        out_hbm.at[c, pl.ds(rbase, OUT_ROWS)])


@functools.partial(
    pl.kernel,
    out_type=(
        jax.ShapeDtypeStruct((NC, N, HID), jnp.float32),
        jax.ShapeDtypeStruct((NC, N, HID), jnp.float32),
    ),
    mesh=_mesh,
    scratch_types=[
        pltpu.VMEM((OUT_ROWS, HID), jnp.float32),
        pltpu.VMEM((OUT_ROWS, HID), jnp.float32),
        pltpu.VMEM((OUT_ROWS, HID), jnp.float32),
        pltpu.VMEM((OUT_ROWS, HID), jnp.float32),
        pltpu.VMEM((HID,), jnp.float32),
        pltpu.VMEM((NCHUNK, CHUNK), jnp.int32),
        pltpu.VMEM((NCHUNK, CHUNK), jnp.int32),
        pltpu.VMEM((NBUF, CHUNK, HID), jnp.float32),
        pltpu.VMEM_SHARED((N, HID), jnp.float32),
        [pltpu.SemaphoreType.DMA] * NBUF,
        [pltpu.SemaphoreType.DMA] * NBUF,
    ],
    compiler_params=pltpu.CompilerParams(use_tc_tiling_on_sc=False),
)
def _agg_mid_sc(agg1_hbm, hs1_hbm, disx_hbm, b1_hbm, src_hbm, dst_hbm,
                zeros2_hbm, out_hbm, hs2d_hbm,
                p0_v, p1_v, hs1_v, dis_v, b1_v, src_v, dst_v, rows_v, agg_sh,
                gsems, ssems):
    c = lax.axis_index("c")
    s = lax.axis_index("s")
    rbase = s * OUT_ROWS
    base = (c * NS + s) * NCHUNK
    pltpu.sync_copy(src_hbm.at[pl.ds(base, NCHUNK)], src_v)
    pltpu.sync_copy(dst_hbm.at[pl.ds(base, NCHUNK)], dst_v)

    @pl.when(s < OUT_TILES)
    def _compute():
        pltpu.sync_copy(zeros2_hbm.at[pl.ds(rbase, OUT_ROWS)],
                        agg_sh.at[pl.ds(rbase, OUT_ROWS)])
        pltpu.sync_copy(agg1_hbm.at[0, pl.ds(rbase, OUT_ROWS)], p0_v)
        pltpu.sync_copy(agg1_hbm.at[1, pl.ds(rbase, OUT_ROWS)], p1_v)
        pltpu.sync_copy(hs1_hbm.at[pl.ds(rbase, OUT_ROWS)], hs1_v)
        pltpu.sync_copy(disx_hbm.at[pl.ds(rbase, OUT_ROWS)], dis_v)
        pltpu.sync_copy(b1_hbm, b1_v)
        b1r = b1_v[...]

        def nbody(n, carry):
            dn = dis_v[n]
            row = (p0_v[n] + p1_v[n] + hs1_v[n]) * dn + b1r
            p0_v[n] = jnp.maximum(row, 0.0) * dn
            return carry

        lax.fori_loop(0, OUT_ROWS, nbody, 0)
        pltpu.sync_copy(p0_v, hs2d_hbm.at[c, pl.ds(rbase, OUT_ROWS)])

    plsc.subcore_barrier()

    table = hs2d_hbm.at[c]
    for b in range(NBUF):
        pltpu.async_copy(table.at[src_v.at[b]], rows_v.at[b], gsems[b])

    def body(o, carry):
        for b in range(NBUF):
            m = o * NBUF + b
            pltpu.make_async_copy(
                table.at[src_v.at[m]], rows_v.at[b], gsems[b]).wait()
            pltpu.async_copy(
                rows_v.at[b], agg_sh.at[dst_v.at[m]], ssems[b], add=True)

            @pl.when(o < NCHUNK // NBUF - 1)
            def _next():
                pltpu.make_async_copy(
                    rows_v.at[b], agg_sh.at[dst_v.at[m]], ssems[b]).wait()
                pltpu.async_copy(
                    table.at[src_v.at[m + NBUF]], rows_v.at[b], gsems[b])

        return carry

    lax.fori_loop(0, NCHUNK // NBUF, body, 0)
    for b in range(NBUF):
        m = NCHUNK - NBUF + b
        pltpu.make_async_copy(
            rows_v.at[b], agg_sh.at[dst_v.at[m]], ssems[b]).wait()
    plsc.subcore_barrier()

    @pl.when(s < OUT_TILES)
    def _out():
        pltpu.sync_copy(agg_sh.at[pl.ds(rbase, OUT_ROWS)],
                        out_hbm.at[c, pl.ds(rbase, OUT_ROWS)])


_RB = N
_GRID = N // _RB


def _s1_body(degp_ref, x_ref, w1_ref, hs_ref, dis_ref, disx_ref):
    deg = degp_ref[:, 0] + degp_ref[:, 1] + 1.0
    dis = lax.rsqrt(deg)
    h = jnp.dot(x_ref[...], w1_ref[...], preferred_element_type=jnp.float32)
    hs_ref[...] = h * dis[:, None]
    dis_ref[...] = dis[:, None]
    disx_ref[...] = jnp.broadcast_to(dis[:, None], disx_ref.shape)


def _tc_s1(degp_t, x, W1):
    return pl.pallas_call(
        _s1_body,
        grid=(_GRID,),
        in_specs=[
            pl.BlockSpec((_RB, NC), lambda i: (i, 0)),
            pl.BlockSpec((_RB, D_IN), lambda i: (i, 0)),
            pl.BlockSpec((D_IN, HID), lambda i: (0, 0)),
        ],
        out_specs=[
            pl.BlockSpec((_RB, HID), lambda i: (i, 0)),
            pl.BlockSpec((_RB, 1), lambda i: (i, 0)),
            pl.BlockSpec((_RB, HID), lambda i: (i, 0)),
        ],
        out_shape=[
            jax.ShapeDtypeStruct((N, HID), jnp.float32),
            jax.ShapeDtypeStruct((N, 1), jnp.float32),
            jax.ShapeDtypeStruct((N, HID), jnp.float32),
        ],
    )(degp_t, x, W1)


def _out_body(a_ref, hs2_ref, dis_ref, w2_ref, b2_ref, out_ref):
    a = dis_ref[...] * (a_ref[0] + a_ref[1] + hs2_ref[0])
    o = jnp.dot(a, w2_ref[...], preferred_element_type=jnp.float32)
    o = o + b2_ref[...]
    m = jnp.max(o, axis=1, keepdims=True)
    e = jnp.exp(o - m)
    lse = jnp.log(jnp.sum(e, axis=1, keepdims=True))
    out_ref[...] = (o - m) - lse


def _tc_out(agg2, hs2d, dis, W2, b2):
    return pl.pallas_call(
        _out_body,
        grid=(_GRID,),
        in_specs=[
            pl.BlockSpec((NC, _RB, HID), lambda i: (0, i, 0)),
            pl.BlockSpec((NC, _RB, HID), lambda i: (0, i, 0)),
            pl.BlockSpec((_RB, 1), lambda i: (i, 0)),
            pl.BlockSpec((HID, NUM_CLASSES), lambda i: (0, 0)),
            pl.BlockSpec((1, NUM_CLASSES), lambda i: (0, 0)),
        ],
        out_specs=pl.BlockSpec((_RB, NUM_CLASSES), lambda i: (i, 0)),
        out_shape=jax.ShapeDtypeStruct((N, NUM_CLASSES), jnp.float32),
    )(agg2, hs2d, dis, W2, b2)


def kernel(x, edge_index, W1, b1, W2, b2):
    src2d = edge_index[0].reshape(E // CHUNK, CHUNK)
    dst2d = edge_index[1].reshape(E // CHUNK, CHUNK)
    zeros1 = jnp.zeros((N,), jnp.float32)
    zeros2 = jnp.zeros((N, HID), jnp.float32)

    degp = _deg_sc(dst2d, zeros1)
    hs1, dis, disx = _tc_s1(degp.T, x, W1)
    agg1 = _agg_sc(hs1, src2d, dst2d, zeros2)
    agg2, hs2d = _agg_mid_sc(agg1, hs1, disx, b1, src2d, dst2d, zeros2)
    return _tc_out(agg2, hs2d, dis, W2, b2.reshape(1, NUM_CLASSES))
